# baseline (speedup 1.0000x reference)
import jax
import jax.numpy as jnp
from jax import lax
from jax.experimental import pallas as pl
from jax.experimental.pallas import tpu as pltpu
from jax.experimental.pallas import tpu_sc as plsc

NC = 2
NS = 16
NW = NC * NS

B = 16384
D = 128
CHUNK = 128
B_PER_W = B // NW
NCHUNK = B_PER_W // CHUNK
NSLOT = 3


def _gather_body(pos_hbm, cos_hbm, sin_hbm, cos_out, sin_out,
                 idx_v, cos_buf, sin_buf, *sems):
    sem_gc = sems[0:NSLOT]
    sem_gs = sems[NSLOT:2 * NSLOT]
    sem_wc = sems[2 * NSLOT:3 * NSLOT]
    sem_ws = sems[3 * NSLOT:4 * NSLOT]

    wid = lax.axis_index("s") * NC + lax.axis_index("c")
    row = wid // (4096 // B_PER_W)
    col = (wid % (4096 // B_PER_W)) * B_PER_W
    pltpu.sync_copy(pos_hbm.at[row, pl.ds(col, B_PER_W)], idx_v)

    gc, gs, wc, ws = {}, {}, {}, {}

    def issue_gather(c):
        s = c % NSLOT
        idx = idx_v.at[pl.ds(c * CHUNK, CHUNK)]
        gc[c] = pltpu.async_copy(cos_hbm.at[idx], cos_buf.at[s], sem_gc[s])
        gs[c] = pltpu.async_copy(sin_hbm.at[idx], sin_buf.at[s], sem_gs[s])

    for c in range(min(NSLOT, NCHUNK)):
        issue_gather(c)
    for c in range(NCHUNK):
        s = c % NSLOT
        gc[c].wait()
        gs[c].wait()
        base = wid * B_PER_W + c * CHUNK
        wc[c] = pltpu.async_copy(cos_buf.at[s], cos_out.at[pl.ds(base, CHUNK)], sem_wc[s])
        ws[c] = pltpu.async_copy(sin_buf.at[s], sin_out.at[pl.ds(base, CHUNK)], sem_ws[s])
        nxt = c + NSLOT
        if nxt < NCHUNK:
            wc[c].wait()
            ws[c].wait()
            issue_gather(nxt)
    for c in range(max(0, NCHUNK - NSLOT), NCHUNK):
        wc[c].wait()
        ws[c].wait()


@jax.jit
def _rope_gather(pos, cos_cached, sin_cached):
    mesh = plsc.VectorSubcoreMesh(core_axis_name="c", subcore_axis_name="s")
    out_type = (
        jax.ShapeDtypeStruct((B, D), jnp.float32),
        jax.ShapeDtypeStruct((B, D), jnp.float32),
    )
    scratch = [
        pltpu.VMEM((B_PER_W,), jnp.int32),
        pltpu.VMEM((NSLOT, CHUNK, D), jnp.float32),
        pltpu.VMEM((NSLOT, CHUNK, D), jnp.float32),
    ] + [pltpu.SemaphoreType.DMA] * (4 * NSLOT)
    return pl.kernel(
        _gather_body,
        out_type=out_type,
        mesh=mesh,
        scratch_types=scratch,
    )(pos, cos_cached, sin_cached)


def kernel(x, position_ids, cos_cached, sin_cached):
    cos_flat, sin_flat = _rope_gather(position_ids, cos_cached, sin_cached)
    shape = position_ids.shape + (D,)
    return (cos_flat.reshape(shape).astype(x.dtype),
            sin_flat.reshape(shape).astype(x.dtype))
